# baseline (speedup 1.0000x reference)
import functools

import jax
import jax.numpy as jnp
from jax import lax
from jax.experimental import pallas as pl
from jax.experimental.pallas import tpu as pltpu
from jax.experimental.pallas import tpu_sc as plsc

_H = 20
_BT = 16384
_D = 64
_NC = 2
_NS = 16
_NW = _NC * _NS
_C = _BT // _NW

_mesh = plsc.VectorSubcoreMesh(core_axis_name="c", subcore_axis_name="s")


@functools.partial(
    pl.kernel,
    out_type=jax.ShapeDtypeStruct((_H, _D // 8, _BT // 128, 8, 128), jnp.float32),
    mesh=_mesh,
    scratch_types=[
        pltpu.VMEM((2, _C), jnp.int32),
        pltpu.VMEM((2, _C, _D), jnp.float32),
        pltpu.VMEM((_D // 8, _C // 128, 8, 128), jnp.float32),
        pltpu.SemaphoreType.DMA,
        pltpu.SemaphoreType.DMA,
    ],
    compiler_params=pltpu.CompilerParams(
        use_tc_tiling_on_sc=False, needs_layout_passes=False
    ),
)
def _emb_kernel(idx_hbm, table_hbm, out_hbm, idx_v, rows_v, out_v, gsem, osem):
    wid = lax.axis_index("s") * _NC + lax.axis_index("c")
    b0 = wid * _C
    blk0 = wid * (_C // 128)
    lane = lax.iota(jnp.int32, 16)

    def idx_load(g, slot):
        pltpu.sync_copy(idx_hbm.at[pl.ds(g * _BT + b0, _C)], idx_v.at[slot])

    def gather(g, slot):
        del g
        return pltpu.async_copy(
            table_hbm.at[idx_v.at[slot]], rows_v.at[slot], gsem
        )

    def out_copy(g):
        return pltpu.make_async_copy(
            out_v,
            out_hbm.at[g, :, pl.ds(blk0, _C // 128), :, :],
            osem,
        )

    idx_load(0, 0)
    gather(0, 0)

    def chunk(g, _):
        slot = lax.rem(g, 2)
        nslot = 1 - slot

        @pl.when(g < _H - 1)
        def _prefetch():
            idx_load(g + 1, nslot)
            gather(g + 1, nslot)

        pltpu.make_async_copy(
            table_hbm.at[idx_v.at[slot]], rows_v.at[slot], gsem
        ).wait()

        @pl.when(g > 0)
        def _drain():
            out_copy(g - 1).wait()

        slot_splat = jnp.full((16,), slot, jnp.int32)

        def tb_body(i, _):
            t = i // (_C // 128)
            blk = lax.rem(i, _C // 128)
            row_base = blk * 128 + lane
            for r in range(8):
                col = jnp.full((16,), t * 8 + r, jnp.int32)
                for c0 in range(0, 128, 16):
                    val = plsc.load_gather(
                        rows_v, [slot_splat, row_base + c0, col]
                    )
                    out_v[t, blk, r, pl.ds(c0, 16)] = val
            return 0

        lax.fori_loop(0, (_D // 8) * (_C // 128), tb_body, 0)

        out_copy(g).start()
        return 0

    lax.fori_loop(0, _H, chunk, 0)
    out_copy(_H - 1).wait()


def kernel(indices, weight):
    idx_t = indices.astype(jnp.int32).T.reshape(-1)
    p5 = _emb_kernel(idx_t, weight)
    return p5.transpose((2, 4, 0, 1, 3)).reshape(_BT, _H, _D)


# speedup vs baseline: 1.3991x; 1.3991x over previous
import functools

import jax
import jax.numpy as jnp
from jax import lax
from jax.experimental import pallas as pl
from jax.experimental.pallas import tpu as pltpu
from jax.experimental.pallas import tpu_sc as plsc

_H = 20
_BT = 16384
_D = 64
_NC = 2
_NS = 16
_NW = _NC * _NS
_C = _BT // _NW
_NB = _C // 128
_NT = _D // 8
_CP = 129

_mesh = plsc.VectorSubcoreMesh(core_axis_name="c", subcore_axis_name="s")


@functools.partial(
    pl.kernel,
    out_type=jax.ShapeDtypeStruct((_H, _NT, _BT // 128, 8, 128), jnp.float32),
    mesh=_mesh,
    scratch_types=[
        pltpu.VMEM((2, _C), jnp.int32),
        pltpu.VMEM((2, _C, _D), jnp.float32),
        pltpu.VMEM((_NB, _NT, 8, _CP), jnp.float32),
        pltpu.SemaphoreType.DMA,
        pltpu.SemaphoreType.DMA,
    ],
    compiler_params=pltpu.CompilerParams(
        use_tc_tiling_on_sc=False, needs_layout_passes=False
    ),
)
def _emb_kernel(idx_hbm, table_hbm, out_hbm, idx_v, rows_v, out_v, gsem, osem):
    wid = lax.axis_index("s") * _NC + lax.axis_index("c")
    b0 = wid * _C
    blk0 = wid * _NB
    lane = lax.iota(jnp.int32, 16)
    lane_t = lane // 8
    lane_r = lax.rem(lane, 8)

    def idx_load(g, slot):
        pltpu.sync_copy(idx_hbm.at[pl.ds(g * _BT + b0, _C)], idx_v.at[slot])

    def gather(g, slot):
        del g
        return pltpu.async_copy(
            table_hbm.at[idx_v.at[slot]], rows_v.at[slot], gsem
        )

    def out_copy(g, t):
        return pltpu.make_async_copy(
            out_v.at[:, t, :, pl.ds(0, 128)],
            out_hbm.at[g, t, pl.ds(blk0, _NB), :, :],
            osem,
        )

    idx_load(0, 0)
    gather(0, 0)

    def chunk(g, _):
        slot = lax.rem(g, 2)
        nslot = 1 - slot

        @pl.when(g < _H - 1)
        def _prefetch():
            idx_load(g + 1, nslot)
            gather(g + 1, nslot)

        pltpu.make_async_copy(
            table_hbm.at[idx_v.at[slot]], rows_v.at[slot], gsem
        ).wait()

        @pl.when(g > 0)
        def _drain():
            for t in range(_NT):
                out_copy(g - 1, t).wait()

        def b_body(bb, _):
            base_b = jnp.full((16,), bb, jnp.int32)

            def c_body(c, _):
                j = bb * 128 + c
                base_c = jnp.full((16,), c, jnp.int32)
                for d0 in range(0, _D, 16):
                    val = rows_v[slot, j, pl.ds(d0, 16)]
                    plsc.store_scatter(
                        out_v,
                        [base_b, lane_t + (d0 // 8), lane_r, base_c],
                        val,
                    )
                return 0

            lax.fori_loop(0, 128, c_body, 0)
            return 0

        lax.fori_loop(0, _NB, b_body, 0)

        for t in range(_NT):
            out_copy(g, t).start()
        return 0

    lax.fori_loop(0, _H, chunk, 0)
    for t in range(_NT):
        out_copy(_H - 1, t).wait()


def kernel(indices, weight):
    idx_t = indices.astype(jnp.int32).T.reshape(-1)
    p5 = _emb_kernel(idx_t, weight)
    return p5.transpose((2, 4, 0, 1, 3)).reshape(_BT, _H, _D)
